# baseline (speedup 1.0000x reference)
import functools

import jax
import jax.numpy as jnp
from jax import lax
from jax.experimental import pallas as pl
from jax.experimental.pallas import tpu as pltpu
from jax.experimental.pallas import tpu_sc as plsc

N = 10000
E = 320000
D = 128
EDGE_DIM = 16
NUM_GRAPHS = 16

NC = 2
NS = 16
NW = NC * NS
EPW = E // NW
CH = 80
LANES = 16
H1 = 192000
H2 = E - H1

_MESH = plsc.VectorSubcoreMesh(core_axis_name="c", subcore_axis_name="s")


def _wid():
    return lax.axis_index("s") * NC + lax.axis_index("c")


@functools.partial(
    pl.kernel,
    out_type=jax.ShapeDtypeStruct((E,), jnp.float32),
    mesh=_MESH,
    compiler_params=pltpu.CompilerParams(needs_layout_passes=False),
    scratch_types=[
        pltpu.VMEM((N,), jnp.float32),
        pltpu.VMEM((N,), jnp.float32),
        pltpu.VMEM((N,), jnp.float32),
        pltpu.VMEM((EPW,), jnp.int32),
        pltpu.VMEM((EPW,), jnp.int32),
        pltpu.VMEM((EPW,), jnp.float32),
    ],
)
def _sc_dist2(px_hbm, py_hbm, pz_hbm, src_hbm, dst_hbm, out_hbm,
              px, py, pz, sv, dv, ov):
    base = _wid() * EPW
    pltpu.sync_copy(px_hbm, px)
    pltpu.sync_copy(py_hbm, py)
    pltpu.sync_copy(pz_hbm, pz)
    pltpu.sync_copy(src_hbm.at[pl.ds(base, EPW)], sv)
    pltpu.sync_copy(dst_hbm.at[pl.ds(base, EPW)], dv)

    def body(i, carry):
        o = i * LANES
        s16 = sv[pl.ds(o, LANES)]
        d16 = dv[pl.ds(o, LANES)]
        rx = plsc.load_gather(px, [s16]) - plsc.load_gather(px, [d16])
        ry = plsc.load_gather(py, [s16]) - plsc.load_gather(py, [d16])
        rz = plsc.load_gather(pz, [s16]) - plsc.load_gather(pz, [d16])
        ov[pl.ds(o, LANES)] = rx * rx + ry * ry + rz * rz
        return carry

    lax.fori_loop(0, EPW // LANES, body, 0)
    pltpu.sync_copy(ov, out_hbm.at[pl.ds(base, EPW)])


def _make_gather(ne, off):
    epw = ne // NW
    nch = epw // CH

    @functools.partial(
        pl.kernel,
        out_type=jax.ShapeDtypeStruct((ne, D // 2), jnp.int32),
        mesh=_MESH,
        compiler_params=pltpu.CompilerParams(needs_layout_passes=False),
        scratch_types=[
            pltpu.VMEM((epw,), jnp.int32),
            pltpu.VMEM((epw,), jnp.int32),
            pltpu.VMEM((CH, D), jnp.float32),
            pltpu.VMEM((CH, D), jnp.float32),
            pltpu.VMEM((CH, D), jnp.float32),
            pltpu.VMEM((CH, D), jnp.float32),
            pltpu.VMEM((CH, D // 2), jnp.int32),
            pltpu.VMEM((CH, D // 2), jnp.int32),
            pltpu.SemaphoreType.DMA,
            pltpu.SemaphoreType.DMA,
            pltpu.SemaphoreType.DMA,
            pltpu.SemaphoreType.DMA,
        ],
    )
    def gather(a_hbm, b_hbm, src_hbm, dst_hbm, out_hbm,
               sv, dv, ra0, ra1, rb0, rb1, ro0, ro1,
               sg0, sg1, sw0, sw1):
        base = _wid() * epw
        pltpu.sync_copy(src_hbm.at[pl.ds(off + base, epw)], sv)
        pltpu.sync_copy(dst_hbm.at[pl.ds(off + base, epw)], dv)

        def start_gather(c, ra, rb, sg):
            pltpu.async_copy(a_hbm.at[sv.at[pl.ds(c * CH, CH)]], ra, sg)
            pltpu.async_copy(b_hbm.at[dv.at[pl.ds(c * CH, CH)]], rb, sg)

        def wait_gather(ra, rb, sg):
            pltpu.make_async_copy(a_hbm.at[pl.ds(0, CH)], ra, sg).wait()
            pltpu.make_async_copy(b_hbm.at[pl.ds(0, CH)], rb, sg).wait()

        def wait_write(ro, sw):
            pltpu.make_async_copy(ro, out_hbm.at[pl.ds(0, CH)], sw).wait()

        def add(ra, rb, ro):
            def row(r, c2):
                for c in range(D // (2 * LANES)):
                    sl = pl.ds(c * LANES, LANES)
                    sh = pl.ds(D // 2 + c * LANES, LANES)
                    slo = ra[r, sl] + rb[r, sl]
                    shi = ra[r, sh] + rb[r, sh]
                    pk = plsc.pack(slo, shi,
                                   format=plsc.PackFormat.INTERLEAVED)
                    ro[r, pl.ds(c * LANES, LANES)] = plsc.bitcast(
                        pk, jnp.int32)
                return c2

            lax.fori_loop(0, CH, row, 0)

        start_gather(0, ra0, rb0, sg0)
        start_gather(1, ra1, rb1, sg1)

        def slot(k, c, ra, rb, ro, sg, sw):
            wait_gather(ra, rb, sg)

            @pl.when(k > 0)
            def _():
                wait_write(ro, sw)

            add(ra, rb, ro)
            pltpu.async_copy(ro, out_hbm.at[pl.ds(base + c * CH, CH)], sw)

            @pl.when(c + 2 < nch)
            def _():
                start_gather(c + 2, ra, rb, sg)

        def body(k, carry):
            slot(k, 2 * k, ra0, rb0, ro0, sg0, sw0)
            slot(k, 2 * k + 1, ra1, rb1, ro1, sg1, sw1)
            return carry

        lax.fori_loop(0, nch // 2, body, 0)
        if nch % 2:
            slot(jnp.int32(nch // 2), nch - 1, ra0, rb0, ro0, sg0, sw0)
        wait_write(ro0, sw0)
        wait_write(ro1, sw1)

    return gather


_GATHER_KERNELS = {}


def _sc_gather_add(a, b, src, dst, ne, off):
    key = (ne, off)
    if key not in _GATHER_KERNELS:
        _GATHER_KERNELS[key] = _make_gather(ne, off)
    return _GATHER_KERNELS[key](a, b, src, dst)


def _make_scatter(ne):
    epw = ne // NW
    nch = epw // CH

    @functools.partial(
        pl.kernel,
        out_type=jax.ShapeDtypeStruct((NC, N, D), jnp.float32),
        mesh=_MESH,
        compiler_params=pltpu.CompilerParams(needs_layout_passes=False),
        scratch_types=[
            pltpu.VMEM((nch, CH), jnp.int32),
            pltpu.VMEM((CH, D), jnp.float32),
            pltpu.VMEM((CH, D), jnp.float32),
            pltpu.VMEM_SHARED((N, D), jnp.float32),
            pltpu.SemaphoreType.DMA,
            pltpu.SemaphoreType.DMA,
        ],
    )
    def scatter(m2_hbm, dst3_hbm, zero_hbm, out_hbm, idx2, r0, r1, shared,
                sl0, sl1):
        cid = lax.axis_index("c")
        sid = lax.axis_index("s")
        wid = sid * NC + cid
        base = wid * epw
        row0 = sid * 624
        pltpu.sync_copy(zero_hbm.at[pl.ds(row0, 640)],
                        shared.at[pl.ds(row0, 640)])
        pltpu.sync_copy(dst3_hbm.at[wid], idx2)
        plsc.subcore_barrier()

        def start_load(c, r, sl):
            pltpu.async_copy(m2_hbm.at[pl.ds(base + c * CH, CH)], r, sl)

        def wait_load(r, sl):
            pltpu.make_async_copy(m2_hbm.at[pl.ds(0, CH)], r, sl).wait()

        start_load(0, r0, sl0)
        start_load(1, r1, sl1)

        def slot(c, r, sl):
            wait_load(r, sl)
            pltpu.sync_copy(r, shared.at[idx2.at[c]], add=True)

            @pl.when(c + 2 < nch)
            def _():
                start_load(c + 2, r, sl)

        def body(k, carry):
            slot(2 * k, r0, sl0)
            slot(2 * k + 1, r1, sl1)
            return carry

        lax.fori_loop(0, nch // 2, body, 0)
        if nch % 2:
            slot(nch - 1, r0, sl0)
        plsc.subcore_barrier()
        pltpu.sync_copy(shared.at[pl.ds(row0, 640)],
                        out_hbm.at[cid].at[pl.ds(row0, 640)])

    return scatter


_SCATTER_KERNELS = {}


def _sc_scatter(m2, dst3, zeros_nd, ne):
    if ne not in _SCATTER_KERNELS:
        _SCATTER_KERNELS[ne] = _make_scatter(ne)
    return _SCATTER_KERNELS[ne](m2, dst3, zeros_nd)


_BN = 400
_BE = 2000


def _tc_ab(h, w_src, w_dst, be1):
    def body(h_ref, ws_ref, wd_ref, b_ref, a_ref, b_out_ref):
        hb = h_ref[...]
        a_ref[...] = jnp.dot(hb, ws_ref[...],
                             preferred_element_type=jnp.float32)
        b_out_ref[...] = (
            jnp.dot(hb, wd_ref[...], preferred_element_type=jnp.float32)
            + b_ref[...]
        )

    return pl.pallas_call(
        body,
        grid=(N // _BN,),
        in_specs=[
            pl.BlockSpec((_BN, D), lambda i: (i, 0)),
            pl.BlockSpec((D, D), lambda i: (0, 0)),
            pl.BlockSpec((D, D), lambda i: (0, 0)),
            pl.BlockSpec((1, D), lambda i: (0, 0)),
        ],
        out_specs=[
            pl.BlockSpec((_BN, D), lambda i: (i, 0)),
            pl.BlockSpec((_BN, D), lambda i: (i, 0)),
        ],
        out_shape=[
            jax.ShapeDtypeStruct((N, D), jnp.float32),
            jax.ShapeDtypeStruct((N, D), jnp.float32),
        ],
    )(h, w_src, w_dst, be1.reshape(1, D))


def _make_edge(ne, off):
    ob = off // _BE

    def body(g_ref, ea_ref, d2_ref, w16_ref, wd_ref, w2_ref, b2_ref,
             o_ref):
        x = g_ref[...]
        lo = lax.bitcast_convert_type(x << 16, jnp.float32)
        hi = lax.bitcast_convert_type(x & jnp.int32(-65536), jnp.float32)
        gx = jnp.concatenate([lo, hi], axis=1)
        m1 = jax.nn.silu(
            gx
            + jnp.dot(ea_ref[...], w16_ref[...],
                      preferred_element_type=jnp.float32)
            + d2_ref[...] * wd_ref[...]
        )
        o_ref[...] = jax.nn.silu(
            jnp.dot(m1, w2_ref[...], preferred_element_type=jnp.float32)
            + b2_ref[...]
        )

    def call(g, ea, d2, w16, wd, we2, be2):
        return pl.pallas_call(
            body,
            grid=(ne // _BE,),
            in_specs=[
                pl.BlockSpec((_BE, D // 2), lambda i: (i, 0)),
                pl.BlockSpec((_BE, EDGE_DIM), lambda i: (i + ob, 0)),
                pl.BlockSpec((_BE, 1), lambda i: (i + ob, 0)),
                pl.BlockSpec((EDGE_DIM, D), lambda i: (0, 0)),
                pl.BlockSpec((1, D), lambda i: (0, 0)),
                pl.BlockSpec((D, D), lambda i: (0, 0)),
                pl.BlockSpec((1, D), lambda i: (0, 0)),
            ],
            out_specs=pl.BlockSpec((_BE, D), lambda i: (i, 0)),
            out_shape=jax.ShapeDtypeStruct((ne, D), jnp.float32),
        )(g, ea, d2, w16, wd, we2, be2.reshape(1, D))

    return call


_EDGE_KERNELS = {}


def _tc_edge(g, ea, d2, w16, wd, we2, be2, ne, off):
    key = (ne, off)
    if key not in _EDGE_KERNELS:
        _EDGE_KERNELS[key] = _make_edge(ne, off)
    return _EDGE_KERNELS[key](g, ea, d2, w16, wd, we2, be2)


def _node_update(h_ref, pa_ref, pb_ref, wh_ref, wa_ref, b1_ref, w2_ref,
                 b2_ref):
    agg = pa_ref[0] + pa_ref[1] + pb_ref[0] + pb_ref[1]
    u = jax.nn.silu(
        jnp.dot(h_ref[...], wh_ref[...], preferred_element_type=jnp.float32)
        + jnp.dot(agg, wa_ref[...], preferred_element_type=jnp.float32)
        + b1_ref[...]
    )
    return (
        jnp.dot(u, w2_ref[...], preferred_element_type=jnp.float32)
        + b2_ref[...]
    )


_NODE_SPECS = [
    pl.BlockSpec((_BN, D), lambda i: (i, 0)),
    pl.BlockSpec((NC, _BN, D), lambda i: (0, i, 0)),
    pl.BlockSpec((NC, _BN, D), lambda i: (0, i, 0)),
    pl.BlockSpec((D, D), lambda i: (0, 0)),
    pl.BlockSpec((D, D), lambda i: (0, 0)),
    pl.BlockSpec((1, D), lambda i: (0, 0)),
    pl.BlockSpec((D, D), lambda i: (0, 0)),
    pl.BlockSpec((1, D), lambda i: (0, 0)),
]


def _tc_node_ab(h, pa, pb, wh1_h, wh1_a, bh1, wh2, bh2, w_src, w_dst, be1):
    def body(h_ref, pa_ref, pb_ref, wh_ref, wa_ref, b1_ref, w2_ref, b2_ref,
             ws_ref, wd_ref, be_ref, h_out, a_out, b_out):
        hn = _node_update(h_ref, pa_ref, pb_ref, wh_ref, wa_ref, b1_ref,
                          w2_ref, b2_ref)
        h_out[...] = hn
        a_out[...] = jnp.dot(hn, ws_ref[...],
                             preferred_element_type=jnp.float32)
        b_out[...] = (
            jnp.dot(hn, wd_ref[...], preferred_element_type=jnp.float32)
            + be_ref[...]
        )

    return pl.pallas_call(
        body,
        grid=(N // _BN,),
        in_specs=_NODE_SPECS + [
            pl.BlockSpec((D, D), lambda i: (0, 0)),
            pl.BlockSpec((D, D), lambda i: (0, 0)),
            pl.BlockSpec((1, D), lambda i: (0, 0)),
        ],
        out_specs=[
            pl.BlockSpec((_BN, D), lambda i: (i, 0)),
            pl.BlockSpec((_BN, D), lambda i: (i, 0)),
            pl.BlockSpec((_BN, D), lambda i: (i, 0)),
        ],
        out_shape=[
            jax.ShapeDtypeStruct((N, D), jnp.float32),
            jax.ShapeDtypeStruct((N, D), jnp.float32),
            jax.ShapeDtypeStruct((N, D), jnp.float32),
        ],
    )(h, pa, pb, wh1_h, wh1_a, bh1.reshape(1, D), wh2, bh2.reshape(1, D),
      w_src, w_dst, be1.reshape(1, D))


def _tc_node_head(h, pa, pb, wh1_h, wh1_a, bh1, wh2, bh2, batch3, wout,
                  bout):
    grid = N // _BN

    def body(h_ref, pa_ref, pb_ref, wh_ref, wa_ref, b1_ref, w2_ref, b2_ref,
             b_ref, wo_ref, bo_ref, o_ref, sums, counts):
        i = pl.program_id(0)

        @pl.when(i == 0)
        def _():
            sums[...] = jnp.zeros_like(sums)
            counts[...] = jnp.zeros_like(counts)

        hn = _node_update(h_ref, pa_ref, pb_ref, wh_ref, wa_ref, b1_ref,
                          w2_ref, b2_ref)
        b = b_ref[...].reshape(1, _BN)
        gi = lax.broadcasted_iota(jnp.int32, (NUM_GRAPHS, _BN), 0)
        oh = (gi == b).astype(jnp.float32)
        sums[...] = sums[...] + jnp.dot(oh, hn,
                                        preferred_element_type=jnp.float32)
        counts[...] = counts[...] + jnp.sum(oh, axis=1, keepdims=True)

        @pl.when(i == grid - 1)
        def _():
            pooled = sums[...] / jnp.maximum(counts[...], 1.0)
            o_ref[...] = (
                jnp.dot(pooled, wo_ref[...],
                        preferred_element_type=jnp.float32)
                + bo_ref[...]
            )

    return pl.pallas_call(
        body,
        grid=(grid,),
        in_specs=_NODE_SPECS + [
            pl.BlockSpec((1, 1, _BN), lambda i: (i, 0, 0)),
            pl.BlockSpec((D, 1), lambda i: (0, 0)),
            pl.BlockSpec((1, 1), lambda i: (0, 0)),
        ],
        out_specs=pl.BlockSpec((NUM_GRAPHS, 1), lambda i: (0, 0)),
        out_shape=jax.ShapeDtypeStruct((NUM_GRAPHS, 1), jnp.float32),
        scratch_shapes=[
            pltpu.VMEM((NUM_GRAPHS, D), jnp.float32),
            pltpu.VMEM((NUM_GRAPHS, 1), jnp.float32),
        ],
    )(h, pa, pb, wh1_h, wh1_a, bh1.reshape(1, D), wh2, bh2.reshape(1, D),
      batch3, wout, bout.reshape(1, 1))


def kernel(x, pos, edge_index, edge_attr, batch_indices, params):
    src = edge_index[0].astype(jnp.int32)
    dst = edge_index[1].astype(jnp.int32)
    px = pos[:, 0]
    py = pos[:, 1]
    pz = pos[:, 2]

    d2 = _sc_dist2(px, py, pz, src, dst).reshape(E, 1)
    dst3a = dst[:H1].reshape(NW, H1 // NW // CH, CH)
    dst3b = dst[H1:].reshape(NW, H2 // NW // CH, CH)
    zeros_nd = jnp.zeros((N, D), jnp.float32)

    lp0, lp1 = params["layers"]

    def we1_split(lp):
        we1 = lp["We1"]
        return we1[0:D], we1[D:2 * D], we1[2 * D + 1:], we1[2 * D:2 * D + 1]

    w_src0, w_dst0, w16_0, wd_0 = we1_split(lp0)
    w_src1, w_dst1, w16_1, wd_1 = we1_split(lp1)
    batch3 = batch_indices.astype(jnp.int32).reshape(N // _BN, 1, _BN)

    def layer(a, b, w16, wd, we2, be2):
        ga = _sc_gather_add(a, b, src, dst, H1, 0)
        m2a = _tc_edge(ga, edge_attr, d2, w16, wd, we2, be2, H1, 0)
        gb = _sc_gather_add(a, b, src, dst, H2, H1)
        m2b = _tc_edge(gb, edge_attr, d2, w16, wd, we2, be2, H2, H1)
        pa = _sc_scatter(m2a, dst3a, zeros_nd, H1)
        pb = _sc_scatter(m2b, dst3b, zeros_nd, H2)
        return pa, pb

    a, b = _tc_ab(x, w_src0, w_dst0, lp0["be1"])
    pa, pb = layer(a, b, w16_0, wd_0, lp0["We2"], lp0["be2"])
    h, a, b = _tc_node_ab(x, pa, pb, lp0["Wh1"][:D], lp0["Wh1"][D:],
                          lp0["bh1"], lp0["Wh2"], lp0["bh2"],
                          w_src1, w_dst1, lp1["be1"])

    pa, pb = layer(a, b, w16_1, wd_1, lp1["We2"], lp1["be2"])
    return _tc_node_head(h, pa, pb, lp1["Wh1"][:D], lp1["Wh1"][D:],
                         lp1["bh1"], lp1["Wh2"], lp1["bh2"], batch3,
                         params["Wout"], params["bout"])


# speedup vs baseline: 1.1299x; 1.1299x over previous
import functools

import jax
import jax.numpy as jnp
from jax import lax
from jax.experimental import pallas as pl
from jax.experimental.pallas import tpu as pltpu
from jax.experimental.pallas import tpu_sc as plsc

N = 10000
E = 320000
D = 128
EDGE_DIM = 16
NUM_GRAPHS = 16

NC = 2
NS = 16
NW = NC * NS
EPW = E // NW
CH = 80
LANES = 16
H1 = 192000
H2 = E - H1

_MESH = plsc.VectorSubcoreMesh(core_axis_name="c", subcore_axis_name="s")


def _wid():
    return lax.axis_index("s") * NC + lax.axis_index("c")


@functools.partial(
    pl.kernel,
    out_type=jax.ShapeDtypeStruct((E,), jnp.float32),
    mesh=_MESH,
    compiler_params=pltpu.CompilerParams(needs_layout_passes=False),
    scratch_types=[
        pltpu.VMEM((N,), jnp.float32),
        pltpu.VMEM((N,), jnp.float32),
        pltpu.VMEM((N,), jnp.float32),
        pltpu.VMEM((EPW,), jnp.int32),
        pltpu.VMEM((EPW,), jnp.int32),
        pltpu.VMEM((EPW,), jnp.float32),
    ],
)
def _sc_dist2(px_hbm, py_hbm, pz_hbm, src_hbm, dst_hbm, out_hbm,
              px, py, pz, sv, dv, ov):
    base = _wid() * EPW
    pltpu.sync_copy(px_hbm, px)
    pltpu.sync_copy(py_hbm, py)
    pltpu.sync_copy(pz_hbm, pz)
    pltpu.sync_copy(src_hbm.at[pl.ds(base, EPW)], sv)
    pltpu.sync_copy(dst_hbm.at[pl.ds(base, EPW)], dv)

    def body(i, carry):
        o = i * LANES
        s16 = sv[pl.ds(o, LANES)]
        d16 = dv[pl.ds(o, LANES)]
        rx = plsc.load_gather(px, [s16]) - plsc.load_gather(px, [d16])
        ry = plsc.load_gather(py, [s16]) - plsc.load_gather(py, [d16])
        rz = plsc.load_gather(pz, [s16]) - plsc.load_gather(pz, [d16])
        ov[pl.ds(o, LANES)] = rx * rx + ry * ry + rz * rz
        return carry

    lax.fori_loop(0, EPW // LANES, body, 0)
    pltpu.sync_copy(ov, out_hbm.at[pl.ds(base, EPW)])


def _make_gather(ne, off):
    epw = ne // NW
    nch = epw // CH

    @functools.partial(
        pl.kernel,
        out_type=jax.ShapeDtypeStruct((ne, D // 2), jnp.int32),
        mesh=_MESH,
        compiler_params=pltpu.CompilerParams(needs_layout_passes=False),
        scratch_types=[
            pltpu.VMEM((epw,), jnp.int32),
            pltpu.VMEM((epw,), jnp.int32),
            pltpu.VMEM((CH, D), jnp.float32),
            pltpu.VMEM((CH, D), jnp.float32),
            pltpu.VMEM((CH, D), jnp.float32),
            pltpu.VMEM((CH, D), jnp.float32),
            pltpu.VMEM((CH, D // 2), jnp.int32),
            pltpu.VMEM((CH, D // 2), jnp.int32),
            pltpu.SemaphoreType.DMA,
            pltpu.SemaphoreType.DMA,
            pltpu.SemaphoreType.DMA,
            pltpu.SemaphoreType.DMA,
        ],
    )
    def gather(a_hbm, b_hbm, src_hbm, dst_hbm, out_hbm,
               sv, dv, ra0, ra1, rb0, rb1, ro0, ro1,
               sg0, sg1, sw0, sw1):
        base = _wid() * epw
        pltpu.sync_copy(src_hbm.at[pl.ds(off + base, epw)], sv)
        pltpu.sync_copy(dst_hbm.at[pl.ds(off + base, epw)], dv)

        def start_gather(c, ra, rb, sg):
            pltpu.async_copy(a_hbm.at[sv.at[pl.ds(c * CH, CH)]], ra, sg)
            pltpu.async_copy(b_hbm.at[dv.at[pl.ds(c * CH, CH)]], rb, sg)

        def wait_gather(ra, rb, sg):
            pltpu.make_async_copy(a_hbm.at[pl.ds(0, CH)], ra, sg).wait()
            pltpu.make_async_copy(b_hbm.at[pl.ds(0, CH)], rb, sg).wait()

        def wait_write(ro, sw):
            pltpu.make_async_copy(ro, out_hbm.at[pl.ds(0, CH)], sw).wait()

        def add(ra, rb, ro):
            def row(r, c2):
                for c in range(D // (2 * LANES)):
                    sl = pl.ds(c * LANES, LANES)
                    sh = pl.ds(D // 2 + c * LANES, LANES)
                    slo = ra[r, sl] + rb[r, sl]
                    shi = ra[r, sh] + rb[r, sh]
                    pk = plsc.pack(slo, shi,
                                   format=plsc.PackFormat.INTERLEAVED)
                    ro[r, pl.ds(c * LANES, LANES)] = plsc.bitcast(
                        pk, jnp.int32)
                return c2

            lax.fori_loop(0, CH, row, 0)

        start_gather(0, ra0, rb0, sg0)
        start_gather(1, ra1, rb1, sg1)

        def slot(k, c, ra, rb, ro, sg, sw):
            wait_gather(ra, rb, sg)

            @pl.when(k > 0)
            def _():
                wait_write(ro, sw)

            add(ra, rb, ro)
            pltpu.async_copy(ro, out_hbm.at[pl.ds(base + c * CH, CH)], sw)

            @pl.when(c + 2 < nch)
            def _():
                start_gather(c + 2, ra, rb, sg)

        def body(k, carry):
            slot(k, 2 * k, ra0, rb0, ro0, sg0, sw0)
            slot(k, 2 * k + 1, ra1, rb1, ro1, sg1, sw1)
            return carry

        lax.fori_loop(0, nch // 2, body, 0)
        if nch % 2:
            slot(jnp.int32(nch // 2), nch - 1, ra0, rb0, ro0, sg0, sw0)
        wait_write(ro0, sw0)
        wait_write(ro1, sw1)

    return gather


_GATHER_KERNELS = {}


def _sc_gather_add(a, b, src, dst, ne, off):
    key = (ne, off)
    if key not in _GATHER_KERNELS:
        _GATHER_KERNELS[key] = _make_gather(ne, off)
    return _GATHER_KERNELS[key](a, b, src, dst)


def _make_scatter(ne):
    epw = ne // NW
    nch = epw // CH

    @functools.partial(
        pl.kernel,
        out_type=jax.ShapeDtypeStruct((NC, N, D), jnp.float32),
        mesh=_MESH,
        compiler_params=pltpu.CompilerParams(needs_layout_passes=False),
        scratch_types=[
            pltpu.VMEM((nch, CH), jnp.int32),
            pltpu.VMEM((CH, D), jnp.float32),
            pltpu.VMEM((CH, D), jnp.float32),
            pltpu.VMEM_SHARED((N, D), jnp.float32),
            pltpu.SemaphoreType.DMA,
            pltpu.SemaphoreType.DMA,
        ],
    )
    def scatter(m2_hbm, dst3_hbm, zero_hbm, out_hbm, idx2, r0, r1, shared,
                sl0, sl1):
        cid = lax.axis_index("c")
        sid = lax.axis_index("s")
        wid = sid * NC + cid
        base = wid * epw
        row0 = sid * 624
        pltpu.sync_copy(zero_hbm.at[pl.ds(row0, 640)],
                        shared.at[pl.ds(row0, 640)])
        pltpu.sync_copy(dst3_hbm.at[wid], idx2)
        plsc.subcore_barrier()

        def start_load(c, r, sl):
            pltpu.async_copy(m2_hbm.at[pl.ds(base + c * CH, CH)], r, sl)

        def wait_load(r, sl):
            pltpu.make_async_copy(m2_hbm.at[pl.ds(0, CH)], r, sl).wait()

        start_load(0, r0, sl0)
        start_load(1, r1, sl1)

        def slot(c, r, sl):
            wait_load(r, sl)
            pltpu.sync_copy(r, shared.at[idx2.at[c]], add=True)

            @pl.when(c + 2 < nch)
            def _():
                start_load(c + 2, r, sl)

        def body(k, carry):
            slot(2 * k, r0, sl0)
            slot(2 * k + 1, r1, sl1)
            return carry

        lax.fori_loop(0, nch // 2, body, 0)
        if nch % 2:
            slot(nch - 1, r0, sl0)
        plsc.subcore_barrier()
        pltpu.sync_copy(shared.at[pl.ds(row0, 640)],
                        out_hbm.at[cid].at[pl.ds(row0, 640)])

    return scatter


_SCATTER_KERNELS = {}


def _sc_scatter(m2, dst3, zeros_nd, ne):
    if ne not in _SCATTER_KERNELS:
        _SCATTER_KERNELS[ne] = _make_scatter(ne)
    return _SCATTER_KERNELS[ne](m2, dst3, zeros_nd)


_BN = 400
_BE = 2000


def _tc_ab(h, w_src, w_dst, be1):
    def body(h_ref, ws_ref, wd_ref, b_ref, a_ref, b_out_ref):
        hb = h_ref[...]
        a_ref[...] = jnp.dot(hb, ws_ref[...],
                             preferred_element_type=jnp.float32)
        b_out_ref[...] = (
            jnp.dot(hb, wd_ref[...], preferred_element_type=jnp.float32)
            + b_ref[...]
        )

    return pl.pallas_call(
        body,
        grid=(N // _BN,),
        in_specs=[
            pl.BlockSpec((_BN, D), lambda i: (i, 0)),
            pl.BlockSpec((D, D), lambda i: (0, 0)),
            pl.BlockSpec((D, D), lambda i: (0, 0)),
            pl.BlockSpec((1, D), lambda i: (0, 0)),
        ],
        out_specs=[
            pl.BlockSpec((_BN, D), lambda i: (i, 0)),
            pl.BlockSpec((_BN, D), lambda i: (i, 0)),
        ],
        out_shape=[
            jax.ShapeDtypeStruct((N, D), jnp.float32),
            jax.ShapeDtypeStruct((N, D), jnp.float32),
        ],
    )(h, w_src, w_dst, be1.reshape(1, D))


def _make_edge(ne, off):
    ob = off // _BE

    def body(g_ref, ea_ref, w17_ref, w2_ref, b2_ref, o_ref):
        x = g_ref[...]
        lo = lax.bitcast_convert_type(x << 16, jnp.float32)
        hi = lax.bitcast_convert_type(x & jnp.int32(-65536), jnp.float32)
        gx = jnp.concatenate([lo, hi], axis=1)
        m1 = jax.nn.silu(
            gx
            + jnp.dot(ea_ref[...], w17_ref[...],
                      preferred_element_type=jnp.float32)
        )
        o_ref[...] = jax.nn.silu(
            jnp.dot(m1, w2_ref[...], preferred_element_type=jnp.float32)
            + b2_ref[...]
        )

    def call(g, ea17, w17, we2, be2):
        return pl.pallas_call(
            body,
            grid=(ne // _BE,),
            in_specs=[
                pl.BlockSpec((_BE, D // 2), lambda i: (i, 0)),
                pl.BlockSpec((_BE, EDGE_DIM + 1), lambda i: (i + ob, 0)),
                pl.BlockSpec((EDGE_DIM + 1, D), lambda i: (0, 0)),
                pl.BlockSpec((D, D), lambda i: (0, 0)),
                pl.BlockSpec((1, D), lambda i: (0, 0)),
            ],
            out_specs=pl.BlockSpec((_BE, D), lambda i: (i, 0)),
            out_shape=jax.ShapeDtypeStruct((ne, D), jnp.float32),
        )(g, ea17, w17, we2, be2.reshape(1, D))

    return call


_EDGE_KERNELS = {}


def _tc_edge(g, ea17, w17, we2, be2, ne, off):
    key = (ne, off)
    if key not in _EDGE_KERNELS:
        _EDGE_KERNELS[key] = _make_edge(ne, off)
    return _EDGE_KERNELS[key](g, ea17, w17, we2, be2)


def _node_update(h_ref, pa_ref, pb_ref, wh_ref, wa_ref, b1_ref, w2_ref,
                 b2_ref):
    agg = pa_ref[0] + pa_ref[1] + pb_ref[0] + pb_ref[1]
    u = jax.nn.silu(
        jnp.dot(h_ref[...], wh_ref[...], preferred_element_type=jnp.float32)
        + jnp.dot(agg, wa_ref[...], preferred_element_type=jnp.float32)
        + b1_ref[...]
    )
    return (
        jnp.dot(u, w2_ref[...], preferred_element_type=jnp.float32)
        + b2_ref[...]
    )


_NODE_SPECS = [
    pl.BlockSpec((_BN, D), lambda i: (i, 0)),
    pl.BlockSpec((NC, _BN, D), lambda i: (0, i, 0)),
    pl.BlockSpec((NC, _BN, D), lambda i: (0, i, 0)),
    pl.BlockSpec((D, D), lambda i: (0, 0)),
    pl.BlockSpec((D, D), lambda i: (0, 0)),
    pl.BlockSpec((1, D), lambda i: (0, 0)),
    pl.BlockSpec((D, D), lambda i: (0, 0)),
    pl.BlockSpec((1, D), lambda i: (0, 0)),
]


def _tc_node_ab(h, pa, pb, wh1_h, wh1_a, bh1, wh2, bh2, w_src, w_dst, be1):
    def body(h_ref, pa_ref, pb_ref, wh_ref, wa_ref, b1_ref, w2_ref, b2_ref,
             ws_ref, wd_ref, be_ref, h_out, a_out, b_out):
        hn = _node_update(h_ref, pa_ref, pb_ref, wh_ref, wa_ref, b1_ref,
                          w2_ref, b2_ref)
        h_out[...] = hn
        a_out[...] = jnp.dot(hn, ws_ref[...],
                             preferred_element_type=jnp.float32)
        b_out[...] = (
            jnp.dot(hn, wd_ref[...], preferred_element_type=jnp.float32)
            + be_ref[...]
        )

    return pl.pallas_call(
        body,
        grid=(N // _BN,),
        in_specs=_NODE_SPECS + [
            pl.BlockSpec((D, D), lambda i: (0, 0)),
            pl.BlockSpec((D, D), lambda i: (0, 0)),
            pl.BlockSpec((1, D), lambda i: (0, 0)),
        ],
        out_specs=[
            pl.BlockSpec((_BN, D), lambda i: (i, 0)),
            pl.BlockSpec((_BN, D), lambda i: (i, 0)),
            pl.BlockSpec((_BN, D), lambda i: (i, 0)),
        ],
        out_shape=[
            jax.ShapeDtypeStruct((N, D), jnp.float32),
            jax.ShapeDtypeStruct((N, D), jnp.float32),
            jax.ShapeDtypeStruct((N, D), jnp.float32),
        ],
    )(h, pa, pb, wh1_h, wh1_a, bh1.reshape(1, D), wh2, bh2.reshape(1, D),
      w_src, w_dst, be1.reshape(1, D))


def _tc_node_head(h, pa, pb, wh1_h, wh1_a, bh1, wh2, bh2, batch3, wout,
                  bout):
    grid = N // _BN

    def body(h_ref, pa_ref, pb_ref, wh_ref, wa_ref, b1_ref, w2_ref, b2_ref,
             b_ref, wo_ref, bo_ref, o_ref, sums, counts):
        i = pl.program_id(0)

        @pl.when(i == 0)
        def _():
            sums[...] = jnp.zeros_like(sums)
            counts[...] = jnp.zeros_like(counts)

        hn = _node_update(h_ref, pa_ref, pb_ref, wh_ref, wa_ref, b1_ref,
                          w2_ref, b2_ref)
        b = b_ref[...].reshape(1, _BN)
        gi = lax.broadcasted_iota(jnp.int32, (NUM_GRAPHS, _BN), 0)
        oh = (gi == b).astype(jnp.float32)
        sums[...] = sums[...] + jnp.dot(oh, hn,
                                        preferred_element_type=jnp.float32)
        counts[...] = counts[...] + jnp.sum(oh, axis=1, keepdims=True)

        @pl.when(i == grid - 1)
        def _():
            pooled = sums[...] / jnp.maximum(counts[...], 1.0)
            o_ref[...] = (
                jnp.dot(pooled, wo_ref[...],
                        preferred_element_type=jnp.float32)
                + bo_ref[...]
            )

    return pl.pallas_call(
        body,
        grid=(grid,),
        in_specs=_NODE_SPECS + [
            pl.BlockSpec((1, 1, _BN), lambda i: (i, 0, 0)),
            pl.BlockSpec((D, 1), lambda i: (0, 0)),
            pl.BlockSpec((1, 1), lambda i: (0, 0)),
        ],
        out_specs=pl.BlockSpec((NUM_GRAPHS, 1), lambda i: (0, 0)),
        out_shape=jax.ShapeDtypeStruct((NUM_GRAPHS, 1), jnp.float32),
        scratch_shapes=[
            pltpu.VMEM((NUM_GRAPHS, D), jnp.float32),
            pltpu.VMEM((NUM_GRAPHS, 1), jnp.float32),
        ],
    )(h, pa, pb, wh1_h, wh1_a, bh1.reshape(1, D), wh2, bh2.reshape(1, D),
      batch3, wout, bout.reshape(1, 1))


def kernel(x, pos, edge_index, edge_attr, batch_indices, params):
    src = edge_index[0].astype(jnp.int32)
    dst = edge_index[1].astype(jnp.int32)
    px = pos[:, 0]
    py = pos[:, 1]
    pz = pos[:, 2]

    d2 = _sc_dist2(px, py, pz, src, dst)
    ea17 = jnp.concatenate([edge_attr, d2[:, None]], axis=1)
    dst3a = dst[:H1].reshape(NW, H1 // NW // CH, CH)
    dst3b = dst[H1:].reshape(NW, H2 // NW // CH, CH)
    zeros_nd = jnp.zeros((N, D), jnp.float32)

    lp0, lp1 = params["layers"]

    def we1_split(lp):
        we1 = lp["We1"]
        w17 = jnp.concatenate([we1[2 * D + 1:], we1[2 * D:2 * D + 1]],
                              axis=0)
        return we1[0:D], we1[D:2 * D], w17

    w_src0, w_dst0, w17_0 = we1_split(lp0)
    w_src1, w_dst1, w17_1 = we1_split(lp1)
    batch3 = batch_indices.astype(jnp.int32).reshape(N // _BN, 1, _BN)

    def layer(a, b, w17, we2, be2):
        ga = _sc_gather_add(a, b, src, dst, H1, 0)
        m2a = _tc_edge(ga, ea17, w17, we2, be2, H1, 0)
        gb = _sc_gather_add(a, b, src, dst, H2, H1)
        m2b = _tc_edge(gb, ea17, w17, we2, be2, H2, H1)
        pa = _sc_scatter(m2a, dst3a, zeros_nd, H1)
        pb = _sc_scatter(m2b, dst3b, zeros_nd, H2)
        return pa, pb

    a, b = _tc_ab(x, w_src0, w_dst0, lp0["be1"])
    pa, pb = layer(a, b, w17_0, lp0["We2"], lp0["be2"])
    h, a, b = _tc_node_ab(x, pa, pb, lp0["Wh1"][:D], lp0["Wh1"][D:],
                          lp0["bh1"], lp0["Wh2"], lp0["bh2"],
                          w_src1, w_dst1, lp1["be1"])

    pa, pb = layer(a, b, w17_1, lp1["We2"], lp1["be2"])
    return _tc_node_head(h, pa, pb, lp1["Wh1"][:D], lp1["Wh1"][D:],
                         lp1["bh1"], lp1["Wh2"], lp1["bh2"], batch3,
                         params["Wout"], params["bout"])


# speedup vs baseline: 1.1830x; 1.0470x over previous
import functools

import jax
import jax.numpy as jnp
from jax import lax
from jax.experimental import pallas as pl
from jax.experimental.pallas import tpu as pltpu
from jax.experimental.pallas import tpu_sc as plsc

N = 10000
E = 320000
D = 128
EDGE_DIM = 16
NUM_GRAPHS = 16

NC = 2
NS = 16
NW = NC * NS
EPW = E // NW
CH = 80
LANES = 16
H1 = 192000
H2 = E - H1

_MESH = plsc.VectorSubcoreMesh(core_axis_name="c", subcore_axis_name="s")


def _wid():
    return lax.axis_index("s") * NC + lax.axis_index("c")


@functools.partial(
    pl.kernel,
    out_type=jax.ShapeDtypeStruct((E,), jnp.float32),
    mesh=_MESH,
    compiler_params=pltpu.CompilerParams(needs_layout_passes=False),
    scratch_types=[
        pltpu.VMEM((N,), jnp.float32),
        pltpu.VMEM((N,), jnp.float32),
        pltpu.VMEM((N,), jnp.float32),
        pltpu.VMEM((EPW,), jnp.int32),
        pltpu.VMEM((EPW,), jnp.int32),
        pltpu.VMEM((EPW,), jnp.float32),
    ],
)
def _sc_dist2(px_hbm, py_hbm, pz_hbm, src_hbm, dst_hbm, out_hbm,
              px, py, pz, sv, dv, ov):
    base = _wid() * EPW
    pltpu.sync_copy(px_hbm, px)
    pltpu.sync_copy(py_hbm, py)
    pltpu.sync_copy(pz_hbm, pz)
    pltpu.sync_copy(src_hbm.at[pl.ds(base, EPW)], sv)
    pltpu.sync_copy(dst_hbm.at[pl.ds(base, EPW)], dv)

    def body(i, carry):
        o = i * LANES
        s16 = sv[pl.ds(o, LANES)]
        d16 = dv[pl.ds(o, LANES)]
        rx = plsc.load_gather(px, [s16]) - plsc.load_gather(px, [d16])
        ry = plsc.load_gather(py, [s16]) - plsc.load_gather(py, [d16])
        rz = plsc.load_gather(pz, [s16]) - plsc.load_gather(pz, [d16])
        ov[pl.ds(o, LANES)] = rx * rx + ry * ry + rz * rz
        return carry

    lax.fori_loop(0, EPW // LANES, body, 0)
    pltpu.sync_copy(ov, out_hbm.at[pl.ds(base, EPW)])


def _make_gather(ne, off):
    epw = ne // NW
    nch = epw // CH

    @functools.partial(
        pl.kernel,
        out_type=jax.ShapeDtypeStruct((ne, D // 2), jnp.int32),
        mesh=_MESH,
        compiler_params=pltpu.CompilerParams(needs_layout_passes=False),
        scratch_types=[
            pltpu.VMEM((epw,), jnp.int32),
            pltpu.VMEM((epw,), jnp.int32),
            pltpu.VMEM((CH, D), jnp.float32),
            pltpu.VMEM((CH, D), jnp.float32),
            pltpu.VMEM((CH, D), jnp.float32),
            pltpu.VMEM((CH, D), jnp.float32),
            pltpu.VMEM((CH, D // 2), jnp.int32),
            pltpu.VMEM((CH, D // 2), jnp.int32),
            pltpu.SemaphoreType.DMA,
            pltpu.SemaphoreType.DMA,
            pltpu.SemaphoreType.DMA,
            pltpu.SemaphoreType.DMA,
        ],
    )
    def gather(a_hbm, b_hbm, src_hbm, dst_hbm, out_hbm,
               sv, dv, ra0, ra1, rb0, rb1, ro0, ro1,
               sg0, sg1, sw0, sw1):
        base = _wid() * epw
        pltpu.sync_copy(src_hbm.at[pl.ds(off + base, epw)], sv)
        pltpu.sync_copy(dst_hbm.at[pl.ds(off + base, epw)], dv)

        def start_gather(c, ra, rb, sg):
            pltpu.async_copy(a_hbm.at[sv.at[pl.ds(c * CH, CH)]], ra, sg)
            pltpu.async_copy(b_hbm.at[dv.at[pl.ds(c * CH, CH)]], rb, sg)

        def wait_gather(ra, rb, sg):
            pltpu.make_async_copy(a_hbm.at[pl.ds(0, CH)], ra, sg).wait()
            pltpu.make_async_copy(b_hbm.at[pl.ds(0, CH)], rb, sg).wait()

        def wait_write(ro, sw):
            pltpu.make_async_copy(ro, out_hbm.at[pl.ds(0, CH)], sw).wait()

        def add(ra, rb, ro):
            def row(r, c2):
                for c in range(D // (2 * LANES)):
                    sl = pl.ds(c * LANES, LANES)
                    sh = pl.ds(D // 2 + c * LANES, LANES)
                    slo = ra[r, sl] + rb[r, sl]
                    shi = ra[r, sh] + rb[r, sh]
                    pk = plsc.pack(slo, shi,
                                   format=plsc.PackFormat.INTERLEAVED)
                    ro[r, pl.ds(c * LANES, LANES)] = plsc.bitcast(
                        pk, jnp.int32)
                return c2

            lax.fori_loop(0, CH, row, 0)

        start_gather(0, ra0, rb0, sg0)
        start_gather(1, ra1, rb1, sg1)

        def slot(k, c, ra, rb, ro, sg, sw):
            wait_gather(ra, rb, sg)

            @pl.when(k > 0)
            def _():
                wait_write(ro, sw)

            add(ra, rb, ro)
            pltpu.async_copy(ro, out_hbm.at[pl.ds(base + c * CH, CH)], sw)

            @pl.when(c + 2 < nch)
            def _():
                start_gather(c + 2, ra, rb, sg)

        def body(k, carry):
            slot(k, 2 * k, ra0, rb0, ro0, sg0, sw0)
            slot(k, 2 * k + 1, ra1, rb1, ro1, sg1, sw1)
            return carry

        lax.fori_loop(0, nch // 2, body, 0)
        if nch % 2:
            slot(jnp.int32(nch // 2), nch - 1, ra0, rb0, ro0, sg0, sw0)
        wait_write(ro0, sw0)
        wait_write(ro1, sw1)

    return gather


_GATHER_KERNELS = {}


def _sc_gather_add(a, b, src, dst, ne, off):
    key = (ne, off)
    if key not in _GATHER_KERNELS:
        _GATHER_KERNELS[key] = _make_gather(ne, off)
    return _GATHER_KERNELS[key](a, b, src, dst)


def _make_scatter(ne):
    epw = ne // NW
    nch = epw // CH

    @functools.partial(
        pl.kernel,
        out_type=jax.ShapeDtypeStruct((NC, N, D), jnp.float32),
        mesh=_MESH,
        compiler_params=pltpu.CompilerParams(needs_layout_passes=False),
        scratch_types=[
            pltpu.VMEM((nch, CH), jnp.int32),
            pltpu.VMEM((CH, D), jnp.float32),
            pltpu.VMEM((CH, D), jnp.float32),
            pltpu.VMEM_SHARED((N, D), jnp.float32),
            pltpu.SemaphoreType.DMA,
            pltpu.SemaphoreType.DMA,
        ],
    )
    def scatter(m2_hbm, dst3_hbm, zero_hbm, out_hbm, idx2, r0, r1, shared,
                sl0, sl1):
        cid = lax.axis_index("c")
        sid = lax.axis_index("s")
        wid = sid * NC + cid
        base = wid * epw
        row0 = sid * 624
        pltpu.sync_copy(zero_hbm.at[pl.ds(row0, 640)],
                        shared.at[pl.ds(row0, 640)])
        pltpu.sync_copy(dst3_hbm.at[wid], idx2)
        plsc.subcore_barrier()

        def start_load(c, r, sl):
            pltpu.async_copy(m2_hbm.at[pl.ds(base + c * CH, CH)], r, sl)

        def wait_load(r, sl):
            pltpu.make_async_copy(m2_hbm.at[pl.ds(0, CH)], r, sl).wait()

        start_load(0, r0, sl0)
        start_load(1, r1, sl1)

        def slot(c, r, sl):
            wait_load(r, sl)
            pltpu.sync_copy(r, shared.at[idx2.at[c]], add=True)

            @pl.when(c + 2 < nch)
            def _():
                start_load(c + 2, r, sl)

        def body(k, carry):
            slot(2 * k, r0, sl0)
            slot(2 * k + 1, r1, sl1)
            return carry

        lax.fori_loop(0, nch // 2, body, 0)
        if nch % 2:
            slot(nch - 1, r0, sl0)
        plsc.subcore_barrier()
        pltpu.sync_copy(shared.at[pl.ds(row0, 640)],
                        out_hbm.at[cid].at[pl.ds(row0, 640)])

    return scatter


_SCATTER_KERNELS = {}


def _sc_scatter(m2, dst3, zeros_nd, ne):
    if ne not in _SCATTER_KERNELS:
        _SCATTER_KERNELS[ne] = _make_scatter(ne)
    return _SCATTER_KERNELS[ne](m2, dst3, zeros_nd)


_BN = 400
_BE = 4000


def _tc_ab(h, w_src, w_dst, be1):
    def body(h_ref, ws_ref, wd_ref, b_ref, a_ref, b_out_ref):
        hb = h_ref[...]
        a_ref[...] = jnp.dot(hb, ws_ref[...],
                             preferred_element_type=jnp.float32)
        b_out_ref[...] = (
            jnp.dot(hb, wd_ref[...], preferred_element_type=jnp.float32)
            + b_ref[...]
        )

    return pl.pallas_call(
        body,
        grid=(N // _BN,),
        in_specs=[
            pl.BlockSpec((_BN, D), lambda i: (i, 0)),
            pl.BlockSpec((D, D), lambda i: (0, 0)),
            pl.BlockSpec((D, D), lambda i: (0, 0)),
            pl.BlockSpec((1, D), lambda i: (0, 0)),
        ],
        out_specs=[
            pl.BlockSpec((_BN, D), lambda i: (i, 0)),
            pl.BlockSpec((_BN, D), lambda i: (i, 0)),
        ],
        out_shape=[
            jax.ShapeDtypeStruct((N, D), jnp.float32),
            jax.ShapeDtypeStruct((N, D), jnp.float32),
        ],
    )(h, w_src, w_dst, be1.reshape(1, D))


def _make_edge(ne, off):
    ob = off // _BE

    def body(g_ref, ea_ref, w17_ref, w2_ref, b2_ref, o_ref):
        x = g_ref[...]
        lo = lax.bitcast_convert_type(x << 16, jnp.float32)
        hi = lax.bitcast_convert_type(x & jnp.int32(-65536), jnp.float32)
        gx = jnp.concatenate([lo, hi], axis=1)
        m1 = jax.nn.silu(
            gx
            + jnp.dot(ea_ref[...], w17_ref[...],
                      preferred_element_type=jnp.float32)
        )
        o_ref[...] = jax.nn.silu(
            jnp.dot(m1, w2_ref[...], preferred_element_type=jnp.float32)
            + b2_ref[...]
        )

    def call(g, ea17, w17, we2, be2):
        return pl.pallas_call(
            body,
            grid=(ne // _BE,),
            in_specs=[
                pl.BlockSpec((_BE, D // 2), lambda i: (i, 0)),
                pl.BlockSpec((_BE, EDGE_DIM + 1), lambda i: (i + ob, 0)),
                pl.BlockSpec((EDGE_DIM + 1, D), lambda i: (0, 0)),
                pl.BlockSpec((D, D), lambda i: (0, 0)),
                pl.BlockSpec((1, D), lambda i: (0, 0)),
            ],
            out_specs=pl.BlockSpec((_BE, D), lambda i: (i, 0)),
            out_shape=jax.ShapeDtypeStruct((ne, D), jnp.float32),
        )(g, ea17, w17, we2, be2.reshape(1, D))

    return call


_EDGE_KERNELS = {}


def _tc_edge(g, ea17, w17, we2, be2, ne, off):
    key = (ne, off)
    if key not in _EDGE_KERNELS:
        _EDGE_KERNELS[key] = _make_edge(ne, off)
    return _EDGE_KERNELS[key](g, ea17, w17, we2, be2)


def _node_update(h_ref, pa_ref, pb_ref, wh_ref, wa_ref, b1_ref, w2_ref,
                 b2_ref):
    agg = pa_ref[0] + pa_ref[1] + pb_ref[0] + pb_ref[1]
    u = jax.nn.silu(
        jnp.dot(h_ref[...], wh_ref[...], preferred_element_type=jnp.float32)
        + jnp.dot(agg, wa_ref[...], preferred_element_type=jnp.float32)
        + b1_ref[...]
    )
    return (
        jnp.dot(u, w2_ref[...], preferred_element_type=jnp.float32)
        + b2_ref[...]
    )


_NODE_SPECS = [
    pl.BlockSpec((_BN, D), lambda i: (i, 0)),
    pl.BlockSpec((NC, _BN, D), lambda i: (0, i, 0)),
    pl.BlockSpec((NC, _BN, D), lambda i: (0, i, 0)),
    pl.BlockSpec((D, D), lambda i: (0, 0)),
    pl.BlockSpec((D, D), lambda i: (0, 0)),
    pl.BlockSpec((1, D), lambda i: (0, 0)),
    pl.BlockSpec((D, D), lambda i: (0, 0)),
    pl.BlockSpec((1, D), lambda i: (0, 0)),
]


def _tc_node_ab(h, pa, pb, wh1_h, wh1_a, bh1, wh2, bh2, w_src, w_dst, be1):
    def body(h_ref, pa_ref, pb_ref, wh_ref, wa_ref, b1_ref, w2_ref, b2_ref,
             ws_ref, wd_ref, be_ref, h_out, a_out, b_out):
        hn = _node_update(h_ref, pa_ref, pb_ref, wh_ref, wa_ref, b1_ref,
                          w2_ref, b2_ref)
        h_out[...] = hn
        a_out[...] = jnp.dot(hn, ws_ref[...],
                             preferred_element_type=jnp.float32)
        b_out[...] = (
            jnp.dot(hn, wd_ref[...], preferred_element_type=jnp.float32)
            + be_ref[...]
        )

    return pl.pallas_call(
        body,
        grid=(N // _BN,),
        in_specs=_NODE_SPECS + [
            pl.BlockSpec((D, D), lambda i: (0, 0)),
            pl.BlockSpec((D, D), lambda i: (0, 0)),
            pl.BlockSpec((1, D), lambda i: (0, 0)),
        ],
        out_specs=[
            pl.BlockSpec((_BN, D), lambda i: (i, 0)),
            pl.BlockSpec((_BN, D), lambda i: (i, 0)),
            pl.BlockSpec((_BN, D), lambda i: (i, 0)),
        ],
        out_shape=[
            jax.ShapeDtypeStruct((N, D), jnp.float32),
            jax.ShapeDtypeStruct((N, D), jnp.float32),
            jax.ShapeDtypeStruct((N, D), jnp.float32),
        ],
    )(h, pa, pb, wh1_h, wh1_a, bh1.reshape(1, D), wh2, bh2.reshape(1, D),
      w_src, w_dst, be1.reshape(1, D))


def _tc_node_head(h, pa, pb, wh1_h, wh1_a, bh1, wh2, bh2, batch3, wout,
                  bout):
    grid = N // _BN

    def body(h_ref, pa_ref, pb_ref, wh_ref, wa_ref, b1_ref, w2_ref, b2_ref,
             b_ref, wo_ref, bo_ref, o_ref, sums, counts):
        i = pl.program_id(0)

        @pl.when(i == 0)
        def _():
            sums[...] = jnp.zeros_like(sums)
            counts[...] = jnp.zeros_like(counts)

        hn = _node_update(h_ref, pa_ref, pb_ref, wh_ref, wa_ref, b1_ref,
                          w2_ref, b2_ref)
        b = b_ref[...].reshape(1, _BN)
        gi = lax.broadcasted_iota(jnp.int32, (NUM_GRAPHS, _BN), 0)
        oh = (gi == b).astype(jnp.float32)
        sums[...] = sums[...] + jnp.dot(oh, hn,
                                        preferred_element_type=jnp.float32)
        counts[...] = counts[...] + jnp.sum(oh, axis=1, keepdims=True)

        @pl.when(i == grid - 1)
        def _():
            pooled = sums[...] / jnp.maximum(counts[...], 1.0)
            o_ref[...] = (
                jnp.dot(pooled, wo_ref[...],
                        preferred_element_type=jnp.float32)
                + bo_ref[...]
            )

    return pl.pallas_call(
        body,
        grid=(grid,),
        in_specs=_NODE_SPECS + [
            pl.BlockSpec((1, 1, _BN), lambda i: (i, 0, 0)),
            pl.BlockSpec((D, 1), lambda i: (0, 0)),
            pl.BlockSpec((1, 1), lambda i: (0, 0)),
        ],
        out_specs=pl.BlockSpec((NUM_GRAPHS, 1), lambda i: (0, 0)),
        out_shape=jax.ShapeDtypeStruct((NUM_GRAPHS, 1), jnp.float32),
        scratch_shapes=[
            pltpu.VMEM((NUM_GRAPHS, D), jnp.float32),
            pltpu.VMEM((NUM_GRAPHS, 1), jnp.float32),
        ],
    )(h, pa, pb, wh1_h, wh1_a, bh1.reshape(1, D), wh2, bh2.reshape(1, D),
      batch3, wout, bout.reshape(1, 1))


def kernel(x, pos, edge_index, edge_attr, batch_indices, params):
    src = edge_index[0].astype(jnp.int32)
    dst = edge_index[1].astype(jnp.int32)
    px = pos[:, 0]
    py = pos[:, 1]
    pz = pos[:, 2]

    d2 = _sc_dist2(px, py, pz, src, dst)
    ea17 = jnp.concatenate([edge_attr, d2[:, None]], axis=1)
    dst3a = dst[:H1].reshape(NW, H1 // NW // CH, CH)
    dst3b = dst[H1:].reshape(NW, H2 // NW // CH, CH)
    zeros_nd = jnp.zeros((N, D), jnp.float32)

    lp0, lp1 = params["layers"]

    def we1_split(lp):
        we1 = lp["We1"]
        w17 = jnp.concatenate([we1[2 * D + 1:], we1[2 * D:2 * D + 1]],
                              axis=0)
        return we1[0:D], we1[D:2 * D], w17

    w_src0, w_dst0, w17_0 = we1_split(lp0)
    w_src1, w_dst1, w17_1 = we1_split(lp1)
    batch3 = batch_indices.astype(jnp.int32).reshape(N // _BN, 1, _BN)

    def layer(a, b, w17, we2, be2):
        ga = _sc_gather_add(a, b, src, dst, H1, 0)
        m2a = _tc_edge(ga, ea17, w17, we2, be2, H1, 0)
        gb = _sc_gather_add(a, b, src, dst, H2, H1)
        m2b = _tc_edge(gb, ea17, w17, we2, be2, H2, H1)
        pa = _sc_scatter(m2a, dst3a, zeros_nd, H1)
        pb = _sc_scatter(m2b, dst3b, zeros_nd, H2)
        return pa, pb

    a, b = _tc_ab(x, w_src0, w_dst0, lp0["be1"])
    pa, pb = layer(a, b, w17_0, lp0["We2"], lp0["be2"])
    h, a, b = _tc_node_ab(x, pa, pb, lp0["Wh1"][:D], lp0["Wh1"][D:],
                          lp0["bh1"], lp0["Wh2"], lp0["bh2"],
                          w_src1, w_dst1, lp1["be1"])

    pa, pb = layer(a, b, w17_1, lp1["We2"], lp1["be2"])
    return _tc_node_head(h, pa, pb, lp1["Wh1"][:D], lp1["Wh1"][D:],
                         lp1["bh1"], lp1["Wh2"], lp1["bh2"], batch3,
                         params["Wout"], params["bout"])


# speedup vs baseline: 1.1972x; 1.0120x over previous
import functools

import jax
import jax.numpy as jnp
from jax import lax
from jax.experimental import pallas as pl
from jax.experimental.pallas import tpu as pltpu
from jax.experimental.pallas import tpu_sc as plsc

N = 10000
E = 320000
D = 128
EDGE_DIM = 16
NUM_GRAPHS = 16

NC = 2
NS = 16
NW = NC * NS
EPW = E // NW
CH = 80
LANES = 16
H1 = 192000
H2 = E - H1

_MESH = plsc.VectorSubcoreMesh(core_axis_name="c", subcore_axis_name="s")


def _wid():
    return lax.axis_index("s") * NC + lax.axis_index("c")


@functools.partial(
    pl.kernel,
    out_type=jax.ShapeDtypeStruct((E,), jnp.float32),
    mesh=_MESH,
    compiler_params=pltpu.CompilerParams(needs_layout_passes=False),
    scratch_types=[
        pltpu.VMEM((N,), jnp.float32),
        pltpu.VMEM((N,), jnp.float32),
        pltpu.VMEM((N,), jnp.float32),
        pltpu.VMEM((EPW,), jnp.int32),
        pltpu.VMEM((EPW,), jnp.int32),
        pltpu.VMEM((EPW,), jnp.float32),
    ],
)
def _sc_dist2(px_hbm, py_hbm, pz_hbm, src_hbm, dst_hbm, out_hbm,
              px, py, pz, sv, dv, ov):
    base = _wid() * EPW
    pltpu.sync_copy(px_hbm, px)
    pltpu.sync_copy(py_hbm, py)
    pltpu.sync_copy(pz_hbm, pz)
    pltpu.sync_copy(src_hbm.at[pl.ds(base, EPW)], sv)
    pltpu.sync_copy(dst_hbm.at[pl.ds(base, EPW)], dv)

    def body(i, carry):
        o = i * LANES
        s16 = sv[pl.ds(o, LANES)]
        d16 = dv[pl.ds(o, LANES)]
        rx = plsc.load_gather(px, [s16]) - plsc.load_gather(px, [d16])
        ry = plsc.load_gather(py, [s16]) - plsc.load_gather(py, [d16])
        rz = plsc.load_gather(pz, [s16]) - plsc.load_gather(pz, [d16])
        ov[pl.ds(o, LANES)] = rx * rx + ry * ry + rz * rz
        return carry

    lax.fori_loop(0, EPW // LANES, body, 0)
    pltpu.sync_copy(ov, out_hbm.at[pl.ds(base, EPW)])


def _make_gather(ne, off):
    epw = ne // NW
    nch = epw // CH

    @functools.partial(
        pl.kernel,
        out_type=jax.ShapeDtypeStruct((ne, D // 2), jnp.int32),
        mesh=_MESH,
        compiler_params=pltpu.CompilerParams(needs_layout_passes=False),
        scratch_types=[
            pltpu.VMEM((epw,), jnp.int32),
            pltpu.VMEM((epw,), jnp.int32),
            pltpu.VMEM((CH, D), jnp.float32),
            pltpu.VMEM((CH, D), jnp.float32),
            pltpu.VMEM((CH, D), jnp.float32),
            pltpu.VMEM((CH, D), jnp.float32),
            pltpu.VMEM((CH, D // 2), jnp.int32),
            pltpu.VMEM((CH, D // 2), jnp.int32),
            pltpu.SemaphoreType.DMA,
            pltpu.SemaphoreType.DMA,
            pltpu.SemaphoreType.DMA,
            pltpu.SemaphoreType.DMA,
        ],
    )
    def gather(a_hbm, b_hbm, src_hbm, dst_hbm, out_hbm,
               sv, dv, ra0, ra1, rb0, rb1, ro0, ro1,
               sg0, sg1, sw0, sw1):
        base = _wid() * epw
        pltpu.sync_copy(src_hbm.at[pl.ds(off + base, epw)], sv)
        pltpu.sync_copy(dst_hbm.at[pl.ds(off + base, epw)], dv)

        def start_gather(c, ra, rb, sg):
            pltpu.async_copy(a_hbm.at[sv.at[pl.ds(c * CH, CH)]], ra, sg)
            pltpu.async_copy(b_hbm.at[dv.at[pl.ds(c * CH, CH)]], rb, sg)

        def wait_gather(ra, rb, sg):
            pltpu.make_async_copy(a_hbm.at[pl.ds(0, CH)], ra, sg).wait()
            pltpu.make_async_copy(b_hbm.at[pl.ds(0, CH)], rb, sg).wait()

        def wait_write(ro, sw):
            pltpu.make_async_copy(ro, out_hbm.at[pl.ds(0, CH)], sw).wait()

        def add(ra, rb, ro):
            def row(r, c2):
                for c in range(D // (2 * LANES)):
                    sl = pl.ds(c * LANES, LANES)
                    sh = pl.ds(D // 2 + c * LANES, LANES)
                    slo = ra[r, sl] + rb[r, sl]
                    shi = ra[r, sh] + rb[r, sh]
                    pk = plsc.pack(slo, shi,
                                   format=plsc.PackFormat.INTERLEAVED)
                    ro[r, pl.ds(c * LANES, LANES)] = plsc.bitcast(
                        pk, jnp.int32)
                return c2

            lax.fori_loop(0, CH, row, 0)

        start_gather(0, ra0, rb0, sg0)
        start_gather(1, ra1, rb1, sg1)

        def slot(k, c, ra, rb, ro, sg, sw):
            wait_gather(ra, rb, sg)

            @pl.when(k > 0)
            def _():
                wait_write(ro, sw)

            add(ra, rb, ro)
            pltpu.async_copy(ro, out_hbm.at[pl.ds(base + c * CH, CH)], sw)

            @pl.when(c + 2 < nch)
            def _():
                start_gather(c + 2, ra, rb, sg)

        def body(k, carry):
            slot(k, 2 * k, ra0, rb0, ro0, sg0, sw0)
            slot(k, 2 * k + 1, ra1, rb1, ro1, sg1, sw1)
            return carry

        lax.fori_loop(0, nch // 2, body, 0)
        if nch % 2:
            slot(jnp.int32(nch // 2), nch - 1, ra0, rb0, ro0, sg0, sw0)
        wait_write(ro0, sw0)
        wait_write(ro1, sw1)

    return gather


_GATHER_KERNELS = {}


def _sc_gather_add(a, b, src, dst, ne, off):
    key = (ne, off)
    if key not in _GATHER_KERNELS:
        _GATHER_KERNELS[key] = _make_gather(ne, off)
    return _GATHER_KERNELS[key](a, b, src, dst)


def _make_scatter(ne):
    epw = ne // NW
    nch = epw // CH

    @functools.partial(
        pl.kernel,
        out_type=jax.ShapeDtypeStruct((NC, N, D), jnp.float32),
        mesh=_MESH,
        compiler_params=pltpu.CompilerParams(needs_layout_passes=False),
        scratch_types=[
            pltpu.VMEM((nch, CH), jnp.int32),
            pltpu.VMEM((CH, D), jnp.float32),
            pltpu.VMEM((CH, D), jnp.float32),
            pltpu.VMEM_SHARED((N, D), jnp.float32),
            pltpu.SemaphoreType.DMA,
            pltpu.SemaphoreType.DMA,
        ],
    )
    def scatter(m2_hbm, dst3_hbm, zero_hbm, out_hbm, idx2, r0, r1, shared,
                sl0, sl1):
        cid = lax.axis_index("c")
        sid = lax.axis_index("s")
        wid = sid * NC + cid
        base = wid * epw
        row0 = sid * 624
        pltpu.sync_copy(zero_hbm.at[pl.ds(row0, 640)],
                        shared.at[pl.ds(row0, 640)])
        pltpu.sync_copy(dst3_hbm.at[wid], idx2)
        plsc.subcore_barrier()

        def start_load(c, r, sl):
            pltpu.async_copy(m2_hbm.at[pl.ds(base + c * CH, CH)], r, sl)

        def wait_load(r, sl):
            pltpu.make_async_copy(m2_hbm.at[pl.ds(0, CH)], r, sl).wait()

        start_load(0, r0, sl0)
        start_load(1, r1, sl1)

        def slot(c, r, sl):
            wait_load(r, sl)
            pltpu.sync_copy(r, shared.at[idx2.at[c]], add=True)

            @pl.when(c + 2 < nch)
            def _():
                start_load(c + 2, r, sl)

        def body(k, carry):
            slot(2 * k, r0, sl0)
            slot(2 * k + 1, r1, sl1)
            return carry

        lax.fori_loop(0, nch // 2, body, 0)
        if nch % 2:
            slot(nch - 1, r0, sl0)
        plsc.subcore_barrier()
        pltpu.sync_copy(shared.at[pl.ds(row0, 640)],
                        out_hbm.at[cid].at[pl.ds(row0, 640)])

    return scatter


_SCATTER_KERNELS = {}


def _sc_scatter(m2, dst3, zeros_nd, ne):
    if ne not in _SCATTER_KERNELS:
        _SCATTER_KERNELS[ne] = _make_scatter(ne)
    return _SCATTER_KERNELS[ne](m2, dst3, zeros_nd)


_BN = 400
_BE = 8000


def _tc_ab(h, w_src, w_dst, be1):
    def body(h_ref, ws_ref, wd_ref, b_ref, a_ref, b_out_ref):
        hb = h_ref[...]
        a_ref[...] = jnp.dot(hb, ws_ref[...],
                             preferred_element_type=jnp.float32)
        b_out_ref[...] = (
            jnp.dot(hb, wd_ref[...], preferred_element_type=jnp.float32)
            + b_ref[...]
        )

    return pl.pallas_call(
        body,
        grid=(N // _BN,),
        in_specs=[
            pl.BlockSpec((_BN, D), lambda i: (i, 0)),
            pl.BlockSpec((D, D), lambda i: (0, 0)),
            pl.BlockSpec((D, D), lambda i: (0, 0)),
            pl.BlockSpec((1, D), lambda i: (0, 0)),
        ],
        out_specs=[
            pl.BlockSpec((_BN, D), lambda i: (i, 0)),
            pl.BlockSpec((_BN, D), lambda i: (i, 0)),
        ],
        out_shape=[
            jax.ShapeDtypeStruct((N, D), jnp.float32),
            jax.ShapeDtypeStruct((N, D), jnp.float32),
        ],
    )(h, w_src, w_dst, be1.reshape(1, D))


def _make_edge(ne, off):
    ob = off // _BE

    def body(g_ref, ea_ref, w17_ref, w2_ref, b2_ref, o_ref):
        x = g_ref[...]
        lo = lax.bitcast_convert_type(x << 16, jnp.float32)
        hi = lax.bitcast_convert_type(x & jnp.int32(-65536), jnp.float32)
        gx = jnp.concatenate([lo, hi], axis=1)
        m1 = jax.nn.silu(
            gx
            + jnp.dot(ea_ref[...], w17_ref[...],
                      preferred_element_type=jnp.float32)
        )
        o_ref[...] = jax.nn.silu(
            jnp.dot(m1, w2_ref[...], preferred_element_type=jnp.float32)
            + b2_ref[...]
        )

    def call(g, ea17, w17, we2, be2):
        return pl.pallas_call(
            body,
            grid=(ne // _BE,),
            in_specs=[
                pl.BlockSpec((_BE, D // 2), lambda i: (i, 0)),
                pl.BlockSpec((_BE, EDGE_DIM + 1), lambda i: (i + ob, 0)),
                pl.BlockSpec((EDGE_DIM + 1, D), lambda i: (0, 0)),
                pl.BlockSpec((D, D), lambda i: (0, 0)),
                pl.BlockSpec((1, D), lambda i: (0, 0)),
            ],
            out_specs=pl.BlockSpec((_BE, D), lambda i: (i, 0)),
            out_shape=jax.ShapeDtypeStruct((ne, D), jnp.float32),
        )(g, ea17, w17, we2, be2.reshape(1, D))

    return call


_EDGE_KERNELS = {}


def _tc_edge(g, ea17, w17, we2, be2, ne, off):
    key = (ne, off)
    if key not in _EDGE_KERNELS:
        _EDGE_KERNELS[key] = _make_edge(ne, off)
    return _EDGE_KERNELS[key](g, ea17, w17, we2, be2)


def _node_update(h_ref, pa_ref, pb_ref, wh_ref, wa_ref, b1_ref, w2_ref,
                 b2_ref):
    agg = pa_ref[0] + pa_ref[1] + pb_ref[0] + pb_ref[1]
    u = jax.nn.silu(
        jnp.dot(h_ref[...], wh_ref[...], preferred_element_type=jnp.float32)
        + jnp.dot(agg, wa_ref[...], preferred_element_type=jnp.float32)
        + b1_ref[...]
    )
    return (
        jnp.dot(u, w2_ref[...], preferred_element_type=jnp.float32)
        + b2_ref[...]
    )


_NODE_SPECS = [
    pl.BlockSpec((_BN, D), lambda i: (i, 0)),
    pl.BlockSpec((NC, _BN, D), lambda i: (0, i, 0)),
    pl.BlockSpec((NC, _BN, D), lambda i: (0, i, 0)),
    pl.BlockSpec((D, D), lambda i: (0, 0)),
    pl.BlockSpec((D, D), lambda i: (0, 0)),
    pl.BlockSpec((1, D), lambda i: (0, 0)),
    pl.BlockSpec((D, D), lambda i: (0, 0)),
    pl.BlockSpec((1, D), lambda i: (0, 0)),
]


def _tc_node_ab(h, pa, pb, wh1_h, wh1_a, bh1, wh2, bh2, w_src, w_dst, be1):
    def body(h_ref, pa_ref, pb_ref, wh_ref, wa_ref, b1_ref, w2_ref, b2_ref,
             ws_ref, wd_ref, be_ref, h_out, a_out, b_out):
        hn = _node_update(h_ref, pa_ref, pb_ref, wh_ref, wa_ref, b1_ref,
                          w2_ref, b2_ref)
        h_out[...] = hn
        a_out[...] = jnp.dot(hn, ws_ref[...],
                             preferred_element_type=jnp.float32)
        b_out[...] = (
            jnp.dot(hn, wd_ref[...], preferred_element_type=jnp.float32)
            + be_ref[...]
        )

    return pl.pallas_call(
        body,
        grid=(N // _BN,),
        in_specs=_NODE_SPECS + [
            pl.BlockSpec((D, D), lambda i: (0, 0)),
            pl.BlockSpec((D, D), lambda i: (0, 0)),
            pl.BlockSpec((1, D), lambda i: (0, 0)),
        ],
        out_specs=[
            pl.BlockSpec((_BN, D), lambda i: (i, 0)),
            pl.BlockSpec((_BN, D), lambda i: (i, 0)),
            pl.BlockSpec((_BN, D), lambda i: (i, 0)),
        ],
        out_shape=[
            jax.ShapeDtypeStruct((N, D), jnp.float32),
            jax.ShapeDtypeStruct((N, D), jnp.float32),
            jax.ShapeDtypeStruct((N, D), jnp.float32),
        ],
    )(h, pa, pb, wh1_h, wh1_a, bh1.reshape(1, D), wh2, bh2.reshape(1, D),
      w_src, w_dst, be1.reshape(1, D))


def _tc_node_head(h, pa, pb, wh1_h, wh1_a, bh1, wh2, bh2, batch3, wout,
                  bout):
    grid = N // _BN

    def body(h_ref, pa_ref, pb_ref, wh_ref, wa_ref, b1_ref, w2_ref, b2_ref,
             b_ref, wo_ref, bo_ref, o_ref, sums, counts):
        i = pl.program_id(0)

        @pl.when(i == 0)
        def _():
            sums[...] = jnp.zeros_like(sums)
            counts[...] = jnp.zeros_like(counts)

        hn = _node_update(h_ref, pa_ref, pb_ref, wh_ref, wa_ref, b1_ref,
                          w2_ref, b2_ref)
        b = b_ref[...].reshape(1, _BN)
        gi = lax.broadcasted_iota(jnp.int32, (NUM_GRAPHS, _BN), 0)
        oh = (gi == b).astype(jnp.float32)
        sums[...] = sums[...] + jnp.dot(oh, hn,
                                        preferred_element_type=jnp.float32)
        counts[...] = counts[...] + jnp.sum(oh, axis=1, keepdims=True)

        @pl.when(i == grid - 1)
        def _():
            pooled = sums[...] / jnp.maximum(counts[...], 1.0)
            o_ref[...] = (
                jnp.dot(pooled, wo_ref[...],
                        preferred_element_type=jnp.float32)
                + bo_ref[...]
            )

    return pl.pallas_call(
        body,
        grid=(grid,),
        in_specs=_NODE_SPECS + [
            pl.BlockSpec((1, 1, _BN), lambda i: (i, 0, 0)),
            pl.BlockSpec((D, 1), lambda i: (0, 0)),
            pl.BlockSpec((1, 1), lambda i: (0, 0)),
        ],
        out_specs=pl.BlockSpec((NUM_GRAPHS, 1), lambda i: (0, 0)),
        out_shape=jax.ShapeDtypeStruct((NUM_GRAPHS, 1), jnp.float32),
        scratch_shapes=[
            pltpu.VMEM((NUM_GRAPHS, D), jnp.float32),
            pltpu.VMEM((NUM_GRAPHS, 1), jnp.float32),
        ],
    )(h, pa, pb, wh1_h, wh1_a, bh1.reshape(1, D), wh2, bh2.reshape(1, D),
      batch3, wout, bout.reshape(1, 1))


def kernel(x, pos, edge_index, edge_attr, batch_indices, params):
    src = edge_index[0].astype(jnp.int32)
    dst = edge_index[1].astype(jnp.int32)
    px = pos[:, 0]
    py = pos[:, 1]
    pz = pos[:, 2]

    d2 = _sc_dist2(px, py, pz, src, dst)
    ea17 = jnp.concatenate([edge_attr, d2[:, None]], axis=1)
    dst3a = dst[:H1].reshape(NW, H1 // NW // CH, CH)
    dst3b = dst[H1:].reshape(NW, H2 // NW // CH, CH)
    zeros_nd = jnp.zeros((N, D), jnp.float32)

    lp0, lp1 = params["layers"]

    def we1_split(lp):
        we1 = lp["We1"]
        w17 = jnp.concatenate([we1[2 * D + 1:], we1[2 * D:2 * D + 1]],
                              axis=0)
        return we1[0:D], we1[D:2 * D], w17

    w_src0, w_dst0, w17_0 = we1_split(lp0)
    w_src1, w_dst1, w17_1 = we1_split(lp1)
    batch3 = batch_indices.astype(jnp.int32).reshape(N // _BN, 1, _BN)

    def layer(a, b, w17, we2, be2):
        ga = _sc_gather_add(a, b, src, dst, H1, 0)
        m2a = _tc_edge(ga, ea17, w17, we2, be2, H1, 0)
        gb = _sc_gather_add(a, b, src, dst, H2, H1)
        m2b = _tc_edge(gb, ea17, w17, we2, be2, H2, H1)
        pa = _sc_scatter(m2a, dst3a, zeros_nd, H1)
        pb = _sc_scatter(m2b, dst3b, zeros_nd, H2)
        return pa, pb

    a, b = _tc_ab(x, w_src0, w_dst0, lp0["be1"])
    pa, pb = layer(a, b, w17_0, lp0["We2"], lp0["be2"])
    h, a, b = _tc_node_ab(x, pa, pb, lp0["Wh1"][:D], lp0["Wh1"][D:],
                          lp0["bh1"], lp0["Wh2"], lp0["bh2"],
                          w_src1, w_dst1, lp1["be1"])

    pa, pb = layer(a, b, w17_1, lp1["We2"], lp1["be2"])
    return _tc_node_head(h, pa, pb, lp1["Wh1"][:D], lp1["Wh1"][D:],
                         lp1["bh1"], lp1["Wh2"], lp1["bh2"], batch3,
                         params["Wout"], params["bout"])


# speedup vs baseline: 1.1986x; 1.0012x over previous
import functools

import jax
import jax.numpy as jnp
from jax import lax
from jax.experimental import pallas as pl
from jax.experimental.pallas import tpu as pltpu
from jax.experimental.pallas import tpu_sc as plsc

N = 10000
E = 320000
D = 128
EDGE_DIM = 16
NUM_GRAPHS = 16

NC = 2
NS = 16
NW = NC * NS
EPW = E // NW
CH = 80
LANES = 16
H1 = 192000
H2 = E - H1

_MESH = plsc.VectorSubcoreMesh(core_axis_name="c", subcore_axis_name="s")


def _wid():
    return lax.axis_index("s") * NC + lax.axis_index("c")


@functools.partial(
    pl.kernel,
    out_type=jax.ShapeDtypeStruct((E,), jnp.float32),
    mesh=_MESH,
    compiler_params=pltpu.CompilerParams(needs_layout_passes=False),
    scratch_types=[
        pltpu.VMEM((N,), jnp.float32),
        pltpu.VMEM((N,), jnp.float32),
        pltpu.VMEM((N,), jnp.float32),
        pltpu.VMEM((EPW,), jnp.int32),
        pltpu.VMEM((EPW,), jnp.int32),
        pltpu.VMEM((EPW,), jnp.float32),
    ],
)
def _sc_dist2(px_hbm, py_hbm, pz_hbm, src_hbm, dst_hbm, out_hbm,
              px, py, pz, sv, dv, ov):
    base = _wid() * EPW
    pltpu.sync_copy(px_hbm, px)
    pltpu.sync_copy(py_hbm, py)
    pltpu.sync_copy(pz_hbm, pz)
    pltpu.sync_copy(src_hbm.at[pl.ds(base, EPW)], sv)
    pltpu.sync_copy(dst_hbm.at[pl.ds(base, EPW)], dv)

    def body(i, carry):
        o = i * LANES
        s16 = sv[pl.ds(o, LANES)]
        d16 = dv[pl.ds(o, LANES)]
        rx = plsc.load_gather(px, [s16]) - plsc.load_gather(px, [d16])
        ry = plsc.load_gather(py, [s16]) - plsc.load_gather(py, [d16])
        rz = plsc.load_gather(pz, [s16]) - plsc.load_gather(pz, [d16])
        ov[pl.ds(o, LANES)] = rx * rx + ry * ry + rz * rz
        return carry

    lax.fori_loop(0, EPW // LANES, body, 0)
    pltpu.sync_copy(ov, out_hbm.at[pl.ds(base, EPW)])


def _make_gather(ne, off):
    epw = ne // NW
    nch = epw // CH

    @functools.partial(
        pl.kernel,
        out_type=jax.ShapeDtypeStruct((ne, D // 2), jnp.int32),
        mesh=_MESH,
        compiler_params=pltpu.CompilerParams(needs_layout_passes=False),
        scratch_types=[
            pltpu.VMEM((epw,), jnp.int32),
            pltpu.VMEM((epw,), jnp.int32),
            pltpu.VMEM((CH, D), jnp.float32),
            pltpu.VMEM((CH, D), jnp.float32),
            pltpu.VMEM((CH, D), jnp.float32),
            pltpu.VMEM((CH, D), jnp.float32),
            pltpu.VMEM((CH, D // 2), jnp.int32),
            pltpu.VMEM((CH, D // 2), jnp.int32),
            pltpu.SemaphoreType.DMA,
            pltpu.SemaphoreType.DMA,
            pltpu.SemaphoreType.DMA,
            pltpu.SemaphoreType.DMA,
        ],
    )
    def gather(a_hbm, b_hbm, src_hbm, dst_hbm, out_hbm,
               sv, dv, ra0, ra1, rb0, rb1, ro0, ro1,
               sg0, sg1, sw0, sw1):
        base = _wid() * epw
        pltpu.sync_copy(src_hbm.at[pl.ds(off + base, epw)], sv)
        pltpu.sync_copy(dst_hbm.at[pl.ds(off + base, epw)], dv)

        def start_gather(c, ra, rb, sg):
            pltpu.async_copy(a_hbm.at[sv.at[pl.ds(c * CH, CH)]], ra, sg)
            pltpu.async_copy(b_hbm.at[dv.at[pl.ds(c * CH, CH)]], rb, sg)

        def wait_gather(ra, rb, sg):
            pltpu.make_async_copy(a_hbm.at[pl.ds(0, CH)], ra, sg).wait()
            pltpu.make_async_copy(b_hbm.at[pl.ds(0, CH)], rb, sg).wait()

        def wait_write(ro, sw):
            pltpu.make_async_copy(ro, out_hbm.at[pl.ds(0, CH)], sw).wait()

        def add(ra, rb, ro):
            def row(r, c2):
                for c in range(D // (2 * LANES)):
                    sl = pl.ds(c * LANES, LANES)
                    sh = pl.ds(D // 2 + c * LANES, LANES)
                    slo = ra[r, sl] + rb[r, sl]
                    shi = ra[r, sh] + rb[r, sh]
                    pk = plsc.pack(slo, shi,
                                   format=plsc.PackFormat.INTERLEAVED)
                    ro[r, pl.ds(c * LANES, LANES)] = plsc.bitcast(
                        pk, jnp.int32)
                return c2

            lax.fori_loop(0, CH, row, 0)

        start_gather(0, ra0, rb0, sg0)
        start_gather(1, ra1, rb1, sg1)

        def slot(k, c, ra, rb, ro, sg, sw):
            wait_gather(ra, rb, sg)

            @pl.when(k > 0)
            def _():
                wait_write(ro, sw)

            add(ra, rb, ro)
            pltpu.async_copy(ro, out_hbm.at[pl.ds(base + c * CH, CH)], sw)

            @pl.when(c + 2 < nch)
            def _():
                start_gather(c + 2, ra, rb, sg)

        def body(k, carry):
            slot(k, 2 * k, ra0, rb0, ro0, sg0, sw0)
            slot(k, 2 * k + 1, ra1, rb1, ro1, sg1, sw1)
            return carry

        lax.fori_loop(0, nch // 2, body, 0)
        if nch % 2:
            slot(jnp.int32(nch // 2), nch - 1, ra0, rb0, ro0, sg0, sw0)
        wait_write(ro0, sw0)
        wait_write(ro1, sw1)

    return gather


_GATHER_KERNELS = {}


def _sc_gather_add(a, b, src, dst, ne, off):
    key = (ne, off)
    if key not in _GATHER_KERNELS:
        _GATHER_KERNELS[key] = _make_gather(ne, off)
    return _GATHER_KERNELS[key](a, b, src, dst)


def _make_scatter(ne):
    epw = ne // NW
    nch = epw // CH

    @functools.partial(
        pl.kernel,
        out_type=jax.ShapeDtypeStruct((NC, N, D), jnp.float32),
        mesh=_MESH,
        compiler_params=pltpu.CompilerParams(needs_layout_passes=False),
        scratch_types=[
            pltpu.VMEM((nch, CH), jnp.int32),
            pltpu.VMEM((CH, D), jnp.float32),
            pltpu.VMEM((CH, D), jnp.float32),
            pltpu.VMEM_SHARED((N, D), jnp.float32),
            pltpu.SemaphoreType.DMA,
            pltpu.SemaphoreType.DMA,
        ],
    )
    def scatter(m2_hbm, dst3_hbm, zero_hbm, out_hbm, idx2, r0, r1, shared,
                sl0, sl1):
        cid = lax.axis_index("c")
        sid = lax.axis_index("s")
        wid = sid * NC + cid
        base = wid * epw
        row0 = sid * 624
        pltpu.sync_copy(zero_hbm.at[pl.ds(row0, 640)],
                        shared.at[pl.ds(row0, 640)])
        pltpu.sync_copy(dst3_hbm.at[wid], idx2)
        plsc.subcore_barrier()

        def start_load(c, r, sl):
            pltpu.async_copy(m2_hbm.at[pl.ds(base + c * CH, CH)], r, sl)

        def wait_load(r, sl):
            pltpu.make_async_copy(m2_hbm.at[pl.ds(0, CH)], r, sl).wait()

        start_load(0, r0, sl0)
        start_load(1, r1, sl1)

        def slot(c, r, sl):
            wait_load(r, sl)
            pltpu.sync_copy(r, shared.at[idx2.at[c]], add=True)

            @pl.when(c + 2 < nch)
            def _():
                start_load(c + 2, r, sl)

        def body(k, carry):
            slot(2 * k, r0, sl0)
            slot(2 * k + 1, r1, sl1)
            return carry

        lax.fori_loop(0, nch // 2, body, 0)
        if nch % 2:
            slot(nch - 1, r0, sl0)
        plsc.subcore_barrier()
        pltpu.sync_copy(shared.at[pl.ds(row0, 640)],
                        out_hbm.at[cid].at[pl.ds(row0, 640)])

    return scatter


_SCATTER_KERNELS = {}


def _sc_scatter(m2, dst3, zeros_nd, ne):
    if ne not in _SCATTER_KERNELS:
        _SCATTER_KERNELS[ne] = _make_scatter(ne)
    return _SCATTER_KERNELS[ne](m2, dst3, zeros_nd)


_BN = 400
_BE = 16000


def _tc_ab(h, w_src, w_dst, be1):
    def body(h_ref, ws_ref, wd_ref, b_ref, a_ref, b_out_ref):
        hb = h_ref[...]
        a_ref[...] = jnp.dot(hb, ws_ref[...],
                             preferred_element_type=jnp.float32)
        b_out_ref[...] = (
            jnp.dot(hb, wd_ref[...], preferred_element_type=jnp.float32)
            + b_ref[...]
        )

    return pl.pallas_call(
        body,
        grid=(N // _BN,),
        in_specs=[
            pl.BlockSpec((_BN, D), lambda i: (i, 0)),
            pl.BlockSpec((D, D), lambda i: (0, 0)),
            pl.BlockSpec((D, D), lambda i: (0, 0)),
            pl.BlockSpec((1, D), lambda i: (0, 0)),
        ],
        out_specs=[
            pl.BlockSpec((_BN, D), lambda i: (i, 0)),
            pl.BlockSpec((_BN, D), lambda i: (i, 0)),
        ],
        out_shape=[
            jax.ShapeDtypeStruct((N, D), jnp.float32),
            jax.ShapeDtypeStruct((N, D), jnp.float32),
        ],
    )(h, w_src, w_dst, be1.reshape(1, D))


def _make_edge(ne, off):
    ob = off // _BE

    def body(g_ref, ea_ref, w17_ref, w2_ref, b2_ref, o_ref):
        x = g_ref[...]
        lo = lax.bitcast_convert_type(x << 16, jnp.float32)
        hi = lax.bitcast_convert_type(x & jnp.int32(-65536), jnp.float32)
        gx = jnp.concatenate([lo, hi], axis=1)
        m1 = jax.nn.silu(
            gx
            + jnp.dot(ea_ref[...], w17_ref[...],
                      preferred_element_type=jnp.float32)
        )
        o_ref[...] = jax.nn.silu(
            jnp.dot(m1, w2_ref[...], preferred_element_type=jnp.float32)
            + b2_ref[...]
        )

    def call(g, ea17, w17, we2, be2):
        return pl.pallas_call(
            body,
            grid=(ne // _BE,),
            in_specs=[
                pl.BlockSpec((_BE, D // 2), lambda i: (i, 0)),
                pl.BlockSpec((_BE, EDGE_DIM + 1), lambda i: (i + ob, 0)),
                pl.BlockSpec((EDGE_DIM + 1, D), lambda i: (0, 0)),
                pl.BlockSpec((D, D), lambda i: (0, 0)),
                pl.BlockSpec((1, D), lambda i: (0, 0)),
            ],
            out_specs=pl.BlockSpec((_BE, D), lambda i: (i, 0)),
            out_shape=jax.ShapeDtypeStruct((ne, D), jnp.float32),
        )(g, ea17, w17, we2, be2.reshape(1, D))

    return call


_EDGE_KERNELS = {}


def _tc_edge(g, ea17, w17, we2, be2, ne, off):
    key = (ne, off)
    if key not in _EDGE_KERNELS:
        _EDGE_KERNELS[key] = _make_edge(ne, off)
    return _EDGE_KERNELS[key](g, ea17, w17, we2, be2)


def _node_update(h_ref, pa_ref, pb_ref, wh_ref, wa_ref, b1_ref, w2_ref,
                 b2_ref):
    agg = pa_ref[0] + pa_ref[1] + pb_ref[0] + pb_ref[1]
    u = jax.nn.silu(
        jnp.dot(h_ref[...], wh_ref[...], preferred_element_type=jnp.float32)
        + jnp.dot(agg, wa_ref[...], preferred_element_type=jnp.float32)
        + b1_ref[...]
    )
    return (
        jnp.dot(u, w2_ref[...], preferred_element_type=jnp.float32)
        + b2_ref[...]
    )


_NODE_SPECS = [
    pl.BlockSpec((_BN, D), lambda i: (i, 0)),
    pl.BlockSpec((NC, _BN, D), lambda i: (0, i, 0)),
    pl.BlockSpec((NC, _BN, D), lambda i: (0, i, 0)),
    pl.BlockSpec((D, D), lambda i: (0, 0)),
    pl.BlockSpec((D, D), lambda i: (0, 0)),
    pl.BlockSpec((1, D), lambda i: (0, 0)),
    pl.BlockSpec((D, D), lambda i: (0, 0)),
    pl.BlockSpec((1, D), lambda i: (0, 0)),
]


def _tc_node_ab(h, pa, pb, wh1_h, wh1_a, bh1, wh2, bh2, w_src, w_dst, be1):
    def body(h_ref, pa_ref, pb_ref, wh_ref, wa_ref, b1_ref, w2_ref, b2_ref,
             ws_ref, wd_ref, be_ref, h_out, a_out, b_out):
        hn = _node_update(h_ref, pa_ref, pb_ref, wh_ref, wa_ref, b1_ref,
                          w2_ref, b2_ref)
        h_out[...] = hn
        a_out[...] = jnp.dot(hn, ws_ref[...],
                             preferred_element_type=jnp.float32)
        b_out[...] = (
            jnp.dot(hn, wd_ref[...], preferred_element_type=jnp.float32)
            + be_ref[...]
        )

    return pl.pallas_call(
        body,
        grid=(N // _BN,),
        in_specs=_NODE_SPECS + [
            pl.BlockSpec((D, D), lambda i: (0, 0)),
            pl.BlockSpec((D, D), lambda i: (0, 0)),
            pl.BlockSpec((1, D), lambda i: (0, 0)),
        ],
        out_specs=[
            pl.BlockSpec((_BN, D), lambda i: (i, 0)),
            pl.BlockSpec((_BN, D), lambda i: (i, 0)),
            pl.BlockSpec((_BN, D), lambda i: (i, 0)),
        ],
        out_shape=[
            jax.ShapeDtypeStruct((N, D), jnp.float32),
            jax.ShapeDtypeStruct((N, D), jnp.float32),
            jax.ShapeDtypeStruct((N, D), jnp.float32),
        ],
    )(h, pa, pb, wh1_h, wh1_a, bh1.reshape(1, D), wh2, bh2.reshape(1, D),
      w_src, w_dst, be1.reshape(1, D))


def _tc_node_head(h, pa, pb, wh1_h, wh1_a, bh1, wh2, bh2, batch3, wout,
                  bout):
    grid = N // _BN

    def body(h_ref, pa_ref, pb_ref, wh_ref, wa_ref, b1_ref, w2_ref, b2_ref,
             b_ref, wo_ref, bo_ref, o_ref, sums, counts):
        i = pl.program_id(0)

        @pl.when(i == 0)
        def _():
            sums[...] = jnp.zeros_like(sums)
            counts[...] = jnp.zeros_like(counts)

        hn = _node_update(h_ref, pa_ref, pb_ref, wh_ref, wa_ref, b1_ref,
                          w2_ref, b2_ref)
        b = b_ref[...].reshape(1, _BN)
        gi = lax.broadcasted_iota(jnp.int32, (NUM_GRAPHS, _BN), 0)
        oh = (gi == b).astype(jnp.float32)
        sums[...] = sums[...] + jnp.dot(oh, hn,
                                        preferred_element_type=jnp.float32)
        counts[...] = counts[...] + jnp.sum(oh, axis=1, keepdims=True)

        @pl.when(i == grid - 1)
        def _():
            pooled = sums[...] / jnp.maximum(counts[...], 1.0)
            o_ref[...] = (
                jnp.dot(pooled, wo_ref[...],
                        preferred_element_type=jnp.float32)
                + bo_ref[...]
            )

    return pl.pallas_call(
        body,
        grid=(grid,),
        in_specs=_NODE_SPECS + [
            pl.BlockSpec((1, 1, _BN), lambda i: (i, 0, 0)),
            pl.BlockSpec((D, 1), lambda i: (0, 0)),
            pl.BlockSpec((1, 1), lambda i: (0, 0)),
        ],
        out_specs=pl.BlockSpec((NUM_GRAPHS, 1), lambda i: (0, 0)),
        out_shape=jax.ShapeDtypeStruct((NUM_GRAPHS, 1), jnp.float32),
        scratch_shapes=[
            pltpu.VMEM((NUM_GRAPHS, D), jnp.float32),
            pltpu.VMEM((NUM_GRAPHS, 1), jnp.float32),
        ],
    )(h, pa, pb, wh1_h, wh1_a, bh1.reshape(1, D), wh2, bh2.reshape(1, D),
      batch3, wout, bout.reshape(1, 1))


def kernel(x, pos, edge_index, edge_attr, batch_indices, params):
    src = edge_index[0].astype(jnp.int32)
    dst = edge_index[1].astype(jnp.int32)
    px = pos[:, 0]
    py = pos[:, 1]
    pz = pos[:, 2]

    d2 = _sc_dist2(px, py, pz, src, dst)
    ea17 = jnp.concatenate([edge_attr, d2[:, None]], axis=1)
    dst3a = dst[:H1].reshape(NW, H1 // NW // CH, CH)
    dst3b = dst[H1:].reshape(NW, H2 // NW // CH, CH)
    zeros_nd = jnp.zeros((N, D), jnp.float32)

    lp0, lp1 = params["layers"]

    def we1_split(lp):
        we1 = lp["We1"]
        w17 = jnp.concatenate([we1[2 * D + 1:], we1[2 * D:2 * D + 1]],
                              axis=0)
        return we1[0:D], we1[D:2 * D], w17

    w_src0, w_dst0, w17_0 = we1_split(lp0)
    w_src1, w_dst1, w17_1 = we1_split(lp1)
    batch3 = batch_indices.astype(jnp.int32).reshape(N // _BN, 1, _BN)

    def layer(a, b, w17, we2, be2):
        ga = _sc_gather_add(a, b, src, dst, H1, 0)
        m2a = _tc_edge(ga, ea17, w17, we2, be2, H1, 0)
        gb = _sc_gather_add(a, b, src, dst, H2, H1)
        m2b = _tc_edge(gb, ea17, w17, we2, be2, H2, H1)
        pa = _sc_scatter(m2a, dst3a, zeros_nd, H1)
        pb = _sc_scatter(m2b, dst3b, zeros_nd, H2)
        return pa, pb

    a, b = _tc_ab(x, w_src0, w_dst0, lp0["be1"])
    pa, pb = layer(a, b, w17_0, lp0["We2"], lp0["be2"])
    h, a, b = _tc_node_ab(x, pa, pb, lp0["Wh1"][:D], lp0["Wh1"][D:],
                          lp0["bh1"], lp0["Wh2"], lp0["bh2"],
                          w_src1, w_dst1, lp1["be1"])

    pa, pb = layer(a, b, w17_1, lp1["We2"], lp1["be2"])
    return _tc_node_head(h, pa, pb, lp1["Wh1"][:D], lp1["Wh1"][D:],
                         lp1["bh1"], lp1["Wh2"], lp1["bh2"], batch3,
                         params["Wout"], params["bout"])
